# baseline (speedup 1.0000x reference)
import functools
import jax
import jax.numpy as jnp
from jax import lax
from jax.experimental import pallas as pl
from jax.experimental.pallas import tpu as pltpu
from jax.experimental.pallas import tpu_sc as plsc

_M = 1048576
_N = 4194304
_NC = 2
_NS = 16
_NW = _NC * _NS
_LANE = 16
_CHUNK = 8192
_ROWW = 128
_UPD_PER_W = _N // _NW
_NCHUNK = _UPD_PER_W // _CHUNK
_STRIPE = _M // _NS
_ZB = 4096
_MH = _M // 2


def _sc_body(idx_hbm, val_hbm, coef_hbm, out_hbm, stage_hbm,
             idx_v, val_v, idx2d, val2d, coef_v, zbuf, acc_sh, sem, sem_ld,
             sem_cb):
    c = lax.axis_index("c")
    s = lax.axis_index("s")
    wid = c * _NS + s

    pltpu.sync_copy(coef_hbm, coef_v)
    a_v = coef_v[pl.ds(0, _LANE)]
    b_v = coef_v[pl.ds(16, _LANE)]
    d_v = coef_v[pl.ds(32, _LANE)]
    h_v = coef_v[pl.ds(48, _LANE)]

    @pl.loop(0, _ZB, step=_LANE)
    def _(i):
        zbuf[pl.ds(i, _LANE)] = h_v

    init_descs = [
        pltpu.async_copy(zbuf, acc_sh.at[pl.ds(s * _STRIPE + q, _ZB)],
                         sem)
        for q in range(0, _STRIPE, _ZB)
    ]
    for dsc in init_descs:
        dsc.wait()

    plsc.subcore_barrier()

    base = wid * _UPD_PER_W
    nrow = _CHUNK // _ROWW

    half = nrow // 2
    drain_bytes_ref = val_v.at[pl.ds(0, half * _ROWW)]
    drain_src = val_hbm.at[pl.ds(0, half * _ROWW)]

    def _compute_rows(j0):
        @pl.loop(j0, j0 + half)
        def _(j):
            for k in range(0, _ROWW, _LANE):
                v = val_v[pl.ds(j * _ROWW + k, _LANE)]
                val2d[j, pl.ds(k, _LANE)] = (a_v * v + b_v) * v + d_v
                idx2d[j, pl.ds(k, _LANE)] = idx_v[pl.ds(j * _ROWW + k, _LANE)]

    def _fire_rows(j0):
        return [
            pltpu.async_copy(val2d.at[j], acc_sh.at[idx2d.at[j]],
                             sem, add=True)
            for j in range(j0, j0 + half)
        ]

    @pl.loop(0, _NCHUNK)
    def _(g):
        e0 = base + g * _CHUNK
        pltpu.sync_copy(idx_hbm.at[pl.ds(e0, _CHUNK)], idx_v)
        pltpu.sync_copy(val_hbm.at[pl.ds(e0, _CHUNK)], val_v)

        _compute_rows(0)

        @pl.when(g > 0)
        def _():
            pltpu.make_async_copy(drain_src, drain_bytes_ref, sem).wait()

        descs_a = _fire_rows(0)
        _compute_rows(half)
        for dsc in descs_a:
            dsc.wait()
        _fire_rows(half)

    pltpu.make_async_copy(drain_src, drain_bytes_ref, sem).wait()

    plsc.subcore_barrier()

    oc = 1 - c
    tchunk = _MH // _NS
    pltpu.sync_copy(acc_sh.at[pl.ds(oc * _MH + s * tchunk, tchunk)],
                    stage_hbm.at[c, pl.ds(s * tchunk, tchunk)])
    plsc.subcore_barrier()

    @pl.when(s == 0)
    def _():
        pltpu.semaphore_signal(sem_cb, 1, core_index=oc)
        pltpu.semaphore_wait(sem_cb, 1)

    plsc.subcore_barrier()

    other = val_v.at[pl.ds(0, _ZB)]

    @pl.loop(0, tchunk, step=_ZB)
    def _(q):
        own_off = c * _MH + s * tchunk + q
        pltpu.sync_copy(acc_sh.at[pl.ds(own_off, _ZB)], zbuf)
        pltpu.sync_copy(stage_hbm.at[oc, pl.ds(s * tchunk + q, _ZB)], other)

        @pl.loop(0, _ZB, step=_LANE)
        def _(i):
            zbuf[pl.ds(i, _LANE)] = jnp.exp(
                zbuf[pl.ds(i, _LANE)] + val_v[pl.ds(i, _LANE)])

        pltpu.sync_copy(zbuf, out_hbm.at[pl.ds(own_off, _ZB)])


@jax.jit
def _sc_accumulate(lst_idx, lst_val, coef):
    mesh = plsc.VectorSubcoreMesh(core_axis_name="c", subcore_axis_name="s")
    f = pl.kernel(
        _sc_body,
        out_type=(jax.ShapeDtypeStruct((_M,), jnp.float32),
                  jax.ShapeDtypeStruct((_NC, _MH), jnp.float32)),
        mesh=mesh,
        scratch_types=[
            pltpu.VMEM((_CHUNK,), jnp.int32),
            pltpu.VMEM((_CHUNK,), jnp.float32),
            pltpu.VMEM((_CHUNK // _ROWW, _ROWW), jnp.int32),
            pltpu.VMEM((_CHUNK // _ROWW, _ROWW), jnp.float32),
            pltpu.VMEM((4 * _LANE,), jnp.float32),
            pltpu.VMEM((_ZB,), jnp.float32),
            pltpu.VMEM_SHARED((_M,), jnp.float32),
            pltpu.SemaphoreType.DMA,
            pltpu.SemaphoreType.DMA,
            pltpu.SemaphoreType.REGULAR,
        ],
    )
    out, _ = f(lst_idx, lst_val, coef)
    return out


def _tc_merge_body(p_ref, o_ref):
    o_ref[...] = jnp.exp(p_ref[0] + p_ref[1])


@jax.jit
def _tc_merge(partial):
    return pl.pallas_call(
        _tc_merge_body,
        grid=(16,),
        in_specs=[pl.BlockSpec((_NC, _M // 16), lambda i: (0, i))],
        out_specs=pl.BlockSpec((_M // 16,), lambda i: (i,)),
        out_shape=jax.ShapeDtypeStruct((_M,), jnp.float32),
    )(partial)


def kernel(batch, lst_idx, lst_val, thetas):
    l5 = jnp.maximum(jnp.minimum(thetas[0], 1.0), 0.0)
    l1 = 1.0 - l5
    inv = 1.0 / thetas[1]
    a = -0.5 * inv * inv
    b = thetas[2] * inv * inv
    d = (jnp.log(l5 * inv) - 0.5 * jnp.log(jnp.float32(2.0 * 3.14159))
         - 0.5 * inv * inv * thetas[2] * thetas[2])
    h = 0.5 * jnp.log(l1)
    coef = jnp.concatenate([
        jnp.full((_LANE,), a, jnp.float32),
        jnp.full((_LANE,), b, jnp.float32),
        jnp.full((_LANE,), d, jnp.float32),
        jnp.full((_LANE,), h, jnp.float32),
    ])

    return _sc_accumulate(lst_idx, lst_val, coef)


# speedup vs baseline: 1.2547x; 1.2547x over previous
import functools
import jax
import jax.numpy as jnp
from jax import lax
from jax.experimental import pallas as pl
from jax.experimental.pallas import tpu as pltpu
from jax.experimental.pallas import tpu_sc as plsc

_M = 1048576
_N = 4194304
_NC = 2
_NS = 16
_NW = _NC * _NS
_LANE = 16
_CHUNK = 8192
_ROWW = 128
_UPD_PER_W = _N // _NW
_NCHUNK = _UPD_PER_W // _CHUNK
_STRIPE = _M // _NS
_ZB = 4096


def _sc_body(idx_hbm, val_hbm, coef_hbm, out_hbm,
             idx_v, val_v, idx2d, val2d, coef_v, zbuf, acc_sh, sem, sem_ld):
    c = lax.axis_index("c")
    s = lax.axis_index("s")
    wid = c * _NS + s

    pltpu.sync_copy(coef_hbm, coef_v)
    a_v = coef_v[pl.ds(0, _LANE)]
    b_v = coef_v[pl.ds(16, _LANE)]
    d_v = coef_v[pl.ds(32, _LANE)]
    h_v = coef_v[pl.ds(48, _LANE)]

    @pl.loop(0, _ZB, step=_LANE)
    def _(i):
        zbuf[pl.ds(i, _LANE)] = h_v

    init_descs = [
        pltpu.async_copy(zbuf, acc_sh.at[pl.ds(s * _STRIPE + q, _ZB)],
                         sem)
        for q in range(0, _STRIPE, _ZB)
    ]
    for dsc in init_descs:
        dsc.wait()

    plsc.subcore_barrier()

    base = wid * _UPD_PER_W
    nrow = _CHUNK // _ROWW

    half = nrow // 2
    drain_bytes_ref = val_v.at[pl.ds(0, half * _ROWW)]
    drain_src = val_hbm.at[pl.ds(0, half * _ROWW)]

    def _compute_rows(j0):
        @pl.loop(j0, j0 + half)
        def _(j):
            for k in range(0, _ROWW, _LANE):
                v = val_v[pl.ds(j * _ROWW + k, _LANE)]
                val2d[j, pl.ds(k, _LANE)] = (a_v * v + b_v) * v + d_v
                idx2d[j, pl.ds(k, _LANE)] = idx_v[pl.ds(j * _ROWW + k, _LANE)]

    def _fire_rows(j0):
        return [
            pltpu.async_copy(val2d.at[j], acc_sh.at[idx2d.at[j]],
                             sem, add=True)
            for j in range(j0, j0 + half)
        ]

    @pl.loop(0, _NCHUNK)
    def _(g):
        e0 = base + g * _CHUNK
        pltpu.sync_copy(idx_hbm.at[pl.ds(e0, _CHUNK)], idx_v)
        pltpu.sync_copy(val_hbm.at[pl.ds(e0, _CHUNK)], val_v)

        _compute_rows(0)

        @pl.when(g > 0)
        def _():
            pltpu.make_async_copy(drain_src, drain_bytes_ref, sem).wait()

        descs_a = _fire_rows(0)
        _compute_rows(half)
        for dsc in descs_a:
            dsc.wait()
        _fire_rows(half)

    pltpu.make_async_copy(drain_src, drain_bytes_ref, sem).wait()

    plsc.subcore_barrier()

    pltpu.sync_copy(acc_sh.at[pl.ds(s * _STRIPE, _STRIPE)],
                    out_hbm.at[c, pl.ds(s * _STRIPE, _STRIPE)])


@jax.jit
def _sc_accumulate(lst_idx, lst_val, coef):
    mesh = plsc.VectorSubcoreMesh(core_axis_name="c", subcore_axis_name="s")
    f = pl.kernel(
        _sc_body,
        out_type=jax.ShapeDtypeStruct((_NC, _M), jnp.float32),
        mesh=mesh,
        scratch_types=[
            pltpu.VMEM((_CHUNK,), jnp.int32),
            pltpu.VMEM((_CHUNK,), jnp.float32),
            pltpu.VMEM((_CHUNK // _ROWW, _ROWW), jnp.int32),
            pltpu.VMEM((_CHUNK // _ROWW, _ROWW), jnp.float32),
            pltpu.VMEM((4 * _LANE,), jnp.float32),
            pltpu.VMEM((_ZB,), jnp.float32),
            pltpu.VMEM_SHARED((_M,), jnp.float32),
            pltpu.SemaphoreType.DMA,
            pltpu.SemaphoreType.DMA,
        ],
    )
    return f(lst_idx, lst_val, coef)


def _tc_merge_body(p_ref, o_ref):
    o_ref[...] = jnp.exp(p_ref[0] + p_ref[1])


@jax.jit
def _tc_merge(partial):
    return pl.pallas_call(
        _tc_merge_body,
        grid=(4,),
        in_specs=[pl.BlockSpec((_NC, _M // 4), lambda i: (0, i))],
        out_specs=pl.BlockSpec((_M // 4,), lambda i: (i,)),
        out_shape=jax.ShapeDtypeStruct((_M,), jnp.float32),
    )(partial)


def kernel(batch, lst_idx, lst_val, thetas):
    l5 = jnp.maximum(jnp.minimum(thetas[0], 1.0), 0.0)
    l1 = 1.0 - l5
    inv = 1.0 / thetas[1]
    a = -0.5 * inv * inv
    b = thetas[2] * inv * inv
    d = (jnp.log(l5 * inv) - 0.5 * jnp.log(jnp.float32(2.0 * 3.14159))
         - 0.5 * inv * inv * thetas[2] * thetas[2])
    h = 0.5 * jnp.log(l1)
    coef = jnp.concatenate([
        jnp.full((_LANE,), a, jnp.float32),
        jnp.full((_LANE,), b, jnp.float32),
        jnp.full((_LANE,), d, jnp.float32),
        jnp.full((_LANE,), h, jnp.float32),
    ])

    partial = _sc_accumulate(lst_idx, lst_val, coef)
    return _tc_merge(partial)


# speedup vs baseline: 1.2725x; 1.0142x over previous
import functools
import jax
import jax.numpy as jnp
from jax import lax
from jax.experimental import pallas as pl
from jax.experimental.pallas import tpu as pltpu
from jax.experimental.pallas import tpu_sc as plsc

_M = 1048576
_N = 4194304
_NC = 2
_NS = 16
_NW = _NC * _NS
_LANE = 16
_CHUNK = 8192
_ROWW = 128
_UPD_PER_W = _N // _NW
_NCHUNK = _UPD_PER_W // _CHUNK
_STRIPE = _M // _NS
_ZB = 4096


def _sc_body(idx_hbm, val_hbm, coef_hbm, out_hbm,
             idx_v, val_v, idx2d, val2d, coef_v, zbuf, acc_sh, sem, sem_ld):
    c = lax.axis_index("c")
    s = lax.axis_index("s")
    wid = c * _NS + s

    pltpu.sync_copy(coef_hbm, coef_v)
    a_v = coef_v[pl.ds(0, _LANE)]
    b_v = coef_v[pl.ds(16, _LANE)]
    d_v = coef_v[pl.ds(32, _LANE)]
    h_v = coef_v[pl.ds(48, _LANE)]

    @pl.loop(0, _ZB, step=_LANE)
    def _(i):
        zbuf[pl.ds(i, _LANE)] = h_v

    init_descs = [
        pltpu.async_copy(zbuf, acc_sh.at[pl.ds(s * _STRIPE + q, _ZB)],
                         sem)
        for q in range(0, _STRIPE, _ZB)
    ]
    for dsc in init_descs:
        dsc.wait()

    plsc.subcore_barrier()

    base = wid * _UPD_PER_W
    nrow = _CHUNK // _ROWW

    half = nrow // 2
    drain_bytes_ref = val_v.at[pl.ds(0, half * _ROWW)]
    drain_src = val_hbm.at[pl.ds(0, half * _ROWW)]

    def _compute_rows(j0):
        @pl.loop(j0, j0 + half)
        def _(j):
            for k in range(0, _ROWW, _LANE):
                v = val_v[pl.ds(j * _ROWW + k, _LANE)]
                val2d[j, pl.ds(k, _LANE)] = (a_v * v + b_v) * v + d_v
                idx2d[j, pl.ds(k, _LANE)] = idx_v[pl.ds(j * _ROWW + k, _LANE)]

    def _fire_rows(j0):
        return [
            pltpu.async_copy(val2d.at[j], acc_sh.at[idx2d.at[j]],
                             sem, add=True)
            for j in range(j0, j0 + half)
        ]

    @pl.loop(0, _NCHUNK)
    def _(g):
        e0 = base + g * _CHUNK
        pltpu.sync_copy(idx_hbm.at[pl.ds(e0, _CHUNK)], idx_v)
        pltpu.sync_copy(val_hbm.at[pl.ds(e0, _CHUNK)], val_v)

        _compute_rows(0)

        @pl.when(g > 0)
        def _():
            pltpu.make_async_copy(drain_src, drain_bytes_ref, sem).wait()

        descs_a = _fire_rows(0)
        _compute_rows(half)
        for dsc in descs_a:
            dsc.wait()
        _fire_rows(half)

    pltpu.make_async_copy(drain_src, drain_bytes_ref, sem).wait()

    plsc.subcore_barrier()

    pltpu.sync_copy(acc_sh.at[pl.ds(s * _STRIPE, _STRIPE)],
                    out_hbm.at[c, pl.ds(s * _STRIPE, _STRIPE)])


@jax.jit
def _sc_accumulate(lst_idx, lst_val, coef):
    mesh = plsc.VectorSubcoreMesh(core_axis_name="c", subcore_axis_name="s")
    f = pl.kernel(
        _sc_body,
        out_type=jax.ShapeDtypeStruct((_NC, _M), jnp.float32),
        mesh=mesh,
        scratch_types=[
            pltpu.VMEM((_CHUNK,), jnp.int32),
            pltpu.VMEM((_CHUNK,), jnp.float32),
            pltpu.VMEM((_CHUNK // _ROWW, _ROWW), jnp.int32),
            pltpu.VMEM((_CHUNK // _ROWW, _ROWW), jnp.float32),
            pltpu.VMEM((4 * _LANE,), jnp.float32),
            pltpu.VMEM((_ZB,), jnp.float32),
            pltpu.VMEM_SHARED((_M,), jnp.float32),
            pltpu.SemaphoreType.DMA,
            pltpu.SemaphoreType.DMA,
        ],
    )
    return f(lst_idx, lst_val, coef)


def _tc_merge_body(p_ref, o_ref):
    o_ref[...] = jnp.exp(p_ref[0] + p_ref[1])


@jax.jit
def _tc_merge(partial):
    return pl.pallas_call(
        _tc_merge_body,
        grid=(2,),
        in_specs=[pl.BlockSpec((_NC, _M // 2), lambda i: (0, i))],
        out_specs=pl.BlockSpec((_M // 2,), lambda i: (i,)),
        out_shape=jax.ShapeDtypeStruct((_M,), jnp.float32),
    )(partial)


def kernel(batch, lst_idx, lst_val, thetas):
    l5 = jnp.maximum(jnp.minimum(thetas[0], 1.0), 0.0)
    l1 = 1.0 - l5
    inv = 1.0 / thetas[1]
    a = -0.5 * inv * inv
    b = thetas[2] * inv * inv
    d = (jnp.log(l5 * inv) - 0.5 * jnp.log(jnp.float32(2.0 * 3.14159))
         - 0.5 * inv * inv * thetas[2] * thetas[2])
    h = 0.5 * jnp.log(l1)
    coef = jnp.concatenate([
        jnp.full((_LANE,), a, jnp.float32),
        jnp.full((_LANE,), b, jnp.float32),
        jnp.full((_LANE,), d, jnp.float32),
        jnp.full((_LANE,), h, jnp.float32),
    ])

    partial = _sc_accumulate(lst_idx, lst_val, coef)
    return _tc_merge(partial)
